# baseline (device time: 504398 ns/iter reference)
import jax
import jax.numpy as jnp
from jax import lax
from jax.experimental import pallas as pl
from jax.experimental.pallas import tpu as pltpu

N_DEV = 16
M = 2048
N = 2048
CHUNK = M // N_DEV
H = 2 * (N_DEV - 1)


def kernel(x, w_mat):
    def body(x_ref, w_ref, out_ref, comm_ref, send_sems, recv_sems, credit_sem):
        me = lax.axis_index("i")
        left = (me + N_DEV - 1) % N_DEV
        right = (me + 1) % N_DEV

        out_ref[:, :] = jnp.dot(
            x_ref[:, :], w_ref[:, :], preferred_element_type=jnp.float32
        )

        barrier_sem = pltpu.get_barrier_semaphore()
        for nbr in (left, right):
            pl.semaphore_signal(
                barrier_sem, inc=1,
                device_id=(nbr,), device_id_type=pl.DeviceIdType.MESH,
            )
        pl.semaphore_wait(barrier_sem, 2)

        comm_ref[0, :, :] = out_ref[pl.ds(me * CHUNK, CHUNK), :]

        for h in range(H):
            send_slot = h % 2
            recv_slot = (h + 1) % 2
            if h >= 1:
                pl.semaphore_wait(credit_sem, 1)
            rdma = pltpu.make_async_remote_copy(
                src_ref=comm_ref.at[send_slot],
                dst_ref=comm_ref.at[recv_slot],
                send_sem=send_sems.at[send_slot],
                recv_sem=recv_sems.at[recv_slot],
                device_id=(right,),
                device_id_type=pl.DeviceIdType.MESH,
            )
            rdma.start()
            rdma.wait()
            if h < H - 1:
                pl.semaphore_signal(
                    credit_sem, inc=1,
                    device_id=(left,), device_id_type=pl.DeviceIdType.MESH,
                )
            if h < N_DEV - 1:
                c = (me + 3 * N_DEV - 1 - h) % N_DEV
                row = c * CHUNK
                acc = comm_ref[recv_slot, :, :] + out_ref[pl.ds(row, CHUNK), :]
                comm_ref[recv_slot, :, :] = acc
                if h == N_DEV - 2:
                    out_ref[pl.ds(row, CHUNK), :] = acc
            else:
                a = h - (N_DEV - 1)
                c = (me + 2 * N_DEV - a) % N_DEV
                out_ref[pl.ds(c * CHUNK, CHUNK), :] = comm_ref[recv_slot, :, :]

    return pl.pallas_call(
        body,
        out_shape=jax.ShapeDtypeStruct((M, N), jnp.float32),
        in_specs=[
            pl.BlockSpec(memory_space=pltpu.VMEM),
            pl.BlockSpec(memory_space=pltpu.VMEM),
        ],
        out_specs=pl.BlockSpec(memory_space=pltpu.VMEM),
        scratch_shapes=[
            pltpu.VMEM((2, CHUNK, N), jnp.float32),
            pltpu.SemaphoreType.DMA((2,)),
            pltpu.SemaphoreType.DMA((2,)),
            pltpu.SemaphoreType.REGULAR,
        ],
        compiler_params=pltpu.CompilerParams(collective_id=0),
    )(x, w_mat)


# device time: 279063 ns/iter; 1.8075x vs baseline; 1.8075x over previous
import jax
import jax.numpy as jnp
from jax import lax
from jax.experimental import pallas as pl
from jax.experimental.pallas import tpu as pltpu

N_DEV = 16
M = 2048
N = 2048
CHUNK = M // N_DEV
HALF = CHUNK // 2
H = 2 * (N_DEV - 1)
S = 4


def kernel(x, w_mat):
    def body(x_ref, w_ref, out_ref,
             comm_f, comm_b,
             send_f, recv_f, send_b, recv_b,
             credit_f, credit_b):
        me = lax.axis_index("i")
        left = (me + N_DEV - 1) % N_DEV
        right = (me + 1) % N_DEV

        out_ref[:, :] = jnp.dot(
            x_ref[:, :], w_ref[:, :], preferred_element_type=jnp.float32
        )

        barrier_sem = pltpu.get_barrier_semaphore()
        for nbr in (left, right):
            pl.semaphore_signal(
                barrier_sem, inc=1,
                device_id=(nbr,), device_id_type=pl.DeviceIdType.MESH,
            )
        pl.semaphore_wait(barrier_sem, 2)

        def top(c):
            return pl.ds(c * CHUNK, HALF)

        def bot(c):
            return pl.ds(c * CHUNK + HALF, HALF)

        comm_f[0, :, :] = out_ref[top(me), :]
        comm_b[0, :, :] = out_ref[bot(me), :]

        def make(h, comm, s_sems, r_sems, dst):
            return pltpu.make_async_remote_copy(
                src_ref=comm.at[h % S],
                dst_ref=comm.at[(h + 1) % S],
                send_sem=s_sems.at[h % S],
                recv_sem=r_sems.at[(h + 1) % S],
                device_id=(dst,),
                device_id_type=pl.DeviceIdType.MESH,
            )

        rf = [make(h, comm_f, send_f, recv_f, right) for h in range(H)]
        rb = [make(h, comm_b, send_b, recv_b, left) for h in range(H)]

        for h in range(H):
            slot = h % S
            if h >= 1:
                rf[h - 1].wait_recv()
                rb[h - 1].wait_recv()
                if h <= N_DEV - 1:
                    cf = (me + 2 * N_DEV - h) % N_DEV
                    cb = (me + h) % N_DEV
                    comm_f[slot, :, :] = comm_f[slot, :, :] + out_ref[top(cf), :]
                    comm_b[slot, :, :] = comm_b[slot, :, :] + out_ref[bot(cb), :]
            if h >= 2:
                rf[h - 2].wait_send()
                rb[h - 2].wait_send()
                if h <= H - 2:
                    pl.semaphore_signal(
                        credit_f, inc=1,
                        device_id=(left,), device_id_type=pl.DeviceIdType.MESH,
                    )
                    pl.semaphore_signal(
                        credit_b, inc=1,
                        device_id=(right,), device_id_type=pl.DeviceIdType.MESH,
                    )
            if h >= 3:
                pl.semaphore_wait(credit_f, 1)
                pl.semaphore_wait(credit_b, 1)
            rf[h].start()
            rb[h].start()
            if h == N_DEV - 1:
                out_ref[top((me + 1) % N_DEV), :] = comm_f[slot, :, :]
                out_ref[bot((me + N_DEV - 1) % N_DEV), :] = comm_b[slot, :, :]
            elif h >= N_DEV:
                a = h - N_DEV
                cf = (me + 2 * N_DEV - a) % N_DEV
                cb = (me + a) % N_DEV
                out_ref[top(cf), :] = comm_f[slot, :, :]
                out_ref[bot(cb), :] = comm_b[slot, :, :]

        rf[H - 1].wait_recv()
        rb[H - 1].wait_recv()
        slot = H % S
        cf = (me + N_DEV + 2) % N_DEV
        cb = (me + N_DEV - 2) % N_DEV
        out_ref[top(cf), :] = comm_f[slot, :, :]
        out_ref[bot(cb), :] = comm_b[slot, :, :]
        for h in (H - 2, H - 1):
            rf[h].wait_send()
            rb[h].wait_send()

    return pl.pallas_call(
        body,
        out_shape=jax.ShapeDtypeStruct((M, N), jnp.float32),
        in_specs=[
            pl.BlockSpec(memory_space=pltpu.VMEM),
            pl.BlockSpec(memory_space=pltpu.VMEM),
        ],
        out_specs=pl.BlockSpec(memory_space=pltpu.VMEM),
        scratch_shapes=[
            pltpu.VMEM((S, HALF, N), jnp.float32),
            pltpu.VMEM((S, HALF, N), jnp.float32),
            pltpu.SemaphoreType.DMA((S,)),
            pltpu.SemaphoreType.DMA((S,)),
            pltpu.SemaphoreType.DMA((S,)),
            pltpu.SemaphoreType.DMA((S,)),
            pltpu.SemaphoreType.REGULAR,
            pltpu.SemaphoreType.REGULAR,
        ],
        compiler_params=pltpu.CompilerParams(collective_id=0),
    )(x, w_mat)


# device time: 187809 ns/iter; 2.6857x vs baseline; 1.4859x over previous
import jax
import jax.numpy as jnp
from jax import lax
from jax.experimental import pallas as pl
from jax.experimental.pallas import tpu as pltpu

N_DEV = 16
M = 2048
N = 2048
CHUNK = M // N_DEV
N_SUB = 4
SUB = CHUNK // N_SUB
H = 2 * (N_DEV - 1)
S = 4


def kernel(x, w_mat):
    def body(x_ref, w_ref, out_ref, *scratch):
        comms = scratch[0:4]
        sems = scratch[4:12]
        credits = scratch[12:16]

        me = lax.axis_index("i")
        left = (me + N_DEV - 1) % N_DEV
        right = (me + 1) % N_DEV

        out_ref[:, :] = jnp.dot(
            x_ref[:, :], w_ref[:, :], preferred_element_type=jnp.float32
        )

        barrier_sem = pltpu.get_barrier_semaphore()
        for nbr in (left, right):
            pl.semaphore_signal(
                barrier_sem, inc=1,
                device_id=(nbr,), device_id_type=pl.DeviceIdType.MESH,
            )
        pl.semaphore_wait(barrier_sem, 2)

        rings = [
            (comms[0], sems[0], sems[1], credits[0], right, left, 0, True),
            (comms[2], sems[4], sems[5], credits[2], left, right, 2, False),
            (comms[1], sems[2], sems[3], credits[1], right, left, 1, True),
            (comms[3], sems[6], sems[7], credits[3], left, right, 3, False),
        ]

        def rows(c, off):
            return pl.ds(c * CHUNK + off * SUB, SUB)

        def chunk_at(h, fwd):
            if fwd:
                return (me + 3 * N_DEV - 1 - h) % N_DEV
            return (me + 1 + h) % N_DEV

        rdmas = []
        for comm, s_sems, r_sems, _, dst, _, _, _ in rings:
            rdmas.append([
                pltpu.make_async_remote_copy(
                    src_ref=comm.at[h % S],
                    dst_ref=comm.at[(h + 1) % S],
                    send_sem=s_sems.at[h % S],
                    recv_sem=r_sems.at[(h + 1) % S],
                    device_id=(dst,),
                    device_id_type=pl.DeviceIdType.MESH,
                )
                for h in range(H)
            ])

        for comm, _, _, _, _, _, off, _ in rings:
            comm[0, :, :] = out_ref[rows(me, off), :]

        for h in range(H):
            slot = h % S
            for r, (comm, _, _, credit, _, credit_dst, off, fwd) in enumerate(rings):
                if h >= 1:
                    rdmas[r][h - 1].wait_recv()
                    if h <= N_DEV - 1:
                        c = chunk_at(h - 1, fwd)
                        comm[slot, :, :] = (
                            comm[slot, :, :] + out_ref[rows(c, off), :]
                        )
                if h >= 2:
                    rdmas[r][h - 2].wait_send()
                    if h <= H - 2:
                        pl.semaphore_signal(
                            credit, inc=1,
                            device_id=(credit_dst,),
                            device_id_type=pl.DeviceIdType.MESH,
                        )
                if h >= 3:
                    pl.semaphore_wait(credit, 1)
                rdmas[r][h].start()
                if h == N_DEV - 1:
                    c = chunk_at(h - 1, fwd)
                    out_ref[rows(c, off), :] = comm[slot, :, :]
                elif h >= N_DEV:
                    a = h - N_DEV
                    c = (me + 2 * N_DEV - a) % N_DEV if fwd else (me + a) % N_DEV
                    out_ref[rows(c, off), :] = comm[slot, :, :]

        slot = H % S
        for r, (comm, _, _, _, _, _, off, fwd) in enumerate(rings):
            rdmas[r][H - 1].wait_recv()
            c = (me + N_DEV + 2) % N_DEV if fwd else (me + N_DEV - 2) % N_DEV
            out_ref[rows(c, off), :] = comm[slot, :, :]
        for r in range(len(rings)):
            rdmas[r][H - 2].wait_send()
            rdmas[r][H - 1].wait_send()

    return pl.pallas_call(
        body,
        out_shape=jax.ShapeDtypeStruct((M, N), jnp.float32),
        in_specs=[
            pl.BlockSpec(memory_space=pltpu.VMEM),
            pl.BlockSpec(memory_space=pltpu.VMEM),
        ],
        out_specs=pl.BlockSpec(memory_space=pltpu.VMEM),
        scratch_shapes=[
            pltpu.VMEM((S, SUB, N), jnp.float32),
            pltpu.VMEM((S, SUB, N), jnp.float32),
            pltpu.VMEM((S, SUB, N), jnp.float32),
            pltpu.VMEM((S, SUB, N), jnp.float32),
            pltpu.SemaphoreType.DMA((S,)), pltpu.SemaphoreType.DMA((S,)),
            pltpu.SemaphoreType.DMA((S,)), pltpu.SemaphoreType.DMA((S,)),
            pltpu.SemaphoreType.DMA((S,)), pltpu.SemaphoreType.DMA((S,)),
            pltpu.SemaphoreType.DMA((S,)), pltpu.SemaphoreType.DMA((S,)),
            pltpu.SemaphoreType.REGULAR, pltpu.SemaphoreType.REGULAR,
            pltpu.SemaphoreType.REGULAR, pltpu.SemaphoreType.REGULAR,
        ],
        compiler_params=pltpu.CompilerParams(collective_id=0),
    )(x, w_mat)
